# Initial kernel scaffold; baseline (speedup 1.0000x reference)
import jax
import jax.numpy as jnp
from jax.experimental import pallas as pl


def kernel(x, edge_index, map_id, node_id, Wp1, bp1, Wl1, bl1, Wr1, Wp2, bp2, Wl2, bl2, Wr2, Wf, bf):
    raise NotImplementedError("write your pallas kernel here")


# speedup vs baseline: 2.8764x; 2.8764x over previous
import functools

import jax
import jax.numpy as jnp
from jax import lax
from jax.experimental import pallas as pl
from jax.experimental.pallas import tpu as pltpu
from jax.experimental.pallas import tpu_sc as plsc

N = 10000
NPAD = 10240
D = 128
E = 320000
CH = 128
NTILES = 32
CPT = 80
NCHUNK = NTILES * CPT
EPAD = NCHUNK * CH
RPT = NPAD // 16
BS = 16
MAP_LEN = 128

_mesh = plsc.VectorSubcoreMesh(
    core_axis_name="c", subcore_axis_name="s", num_cores=2, num_subcores=16)


def _seg_body(with_gather, *refs):
    if with_gather:
        (h_hbm, src_hbm, dst_hbm, z128_hbm,
         acc_out,
         srcv, dstv, rows, acc_sh, isem, gsem) = refs
    else:
        (dst_hbm, z128_hbm, o128_hbm,
         acc_out,
         dstv, rows, acc_sh, isem, gsem) = refs
    c = lax.axis_index("c")
    s = lax.axis_index("s")
    wid = s * 2 + c

    pltpu.sync_copy(z128_hbm, rows)
    for k in range(RPT // CH):
        pltpu.sync_copy(rows, acc_sh.at[pl.ds(s * RPT + k * CH, CH)])
    if not with_gather:
        pltpu.sync_copy(o128_hbm, rows)
    plsc.subcore_barrier()

    def step(j, carry):
        base = (wid * CPT + j) * CH
        if with_gather:
            a = pltpu.async_copy(src_hbm.at[pl.ds(base, CH)], srcv, isem)
            b = pltpu.async_copy(dst_hbm.at[pl.ds(base, CH)], dstv, isem)
            a.wait()
            b.wait()
            pltpu.async_copy(h_hbm.at[srcv], rows, gsem).wait()
        else:
            pltpu.async_copy(dst_hbm.at[pl.ds(base, CH)], dstv, isem).wait()
        pltpu.sync_copy(rows, acc_sh.at[dstv], add=True)
        return carry

    lax.fori_loop(0, CPT, step, 0)
    plsc.subcore_barrier()

    for k in range(RPT // CH):
        r0 = s * RPT + k * CH
        pltpu.sync_copy(acc_sh.at[pl.ds(r0, CH)], rows)
        pltpu.sync_copy(rows, acc_out.at[c, pl.ds(r0, CH)])


def _make_seg(with_gather):
    out_type = jax.ShapeDtypeStruct((2, NPAD, D), jnp.float32)
    scratch = [
        pltpu.VMEM((CH,), jnp.int32),
        pltpu.VMEM((CH,), jnp.int32),
        pltpu.VMEM((CH, D), jnp.float32),
        pltpu.VMEM_SHARED((NPAD, D), jnp.float32),
        pltpu.SemaphoreType.DMA,
        pltpu.SemaphoreType.DMA,
    ]
    if not with_gather:
        del scratch[0]
    return pl.kernel(
        functools.partial(_seg_body, with_gather),
        out_type=out_type,
        mesh=_mesh,
        scratch_types=scratch,
    )


def _gather_body(h_hbm, idx_hbm, out_hbm, idxv, rows, sem):
    c = lax.axis_index("c")
    s = lax.axis_index("s")
    base = s * MAP_LEN + c * 64
    pltpu.sync_copy(idx_hbm.at[pl.ds(base, 64)], idxv)
    pltpu.async_copy(h_hbm.at[idxv], rows, sem).wait()
    pltpu.sync_copy(rows, out_hbm.at[pl.ds(base, 64)])


_local_gather = pl.kernel(
    _gather_body,
    out_type=jax.ShapeDtypeStruct((BS * MAP_LEN, D), jnp.float32),
    mesh=_mesh,
    scratch_types=[
        pltpu.VMEM((64,), jnp.int32),
        pltpu.VMEM((64, D), jnp.float32),
        pltpu.SemaphoreType.DMA,
    ],
)


_BLK = 512
_GRID = NPAD // _BLK


def _row_spec(last=D):
    return pl.BlockSpec((_BLK, last), lambda i: (i, 0))


def _w_spec():
    return pl.BlockSpec((D, D), lambda i: (0, 0))


def _b_spec():
    return pl.BlockSpec((1, D), lambda i: (0, 0))


def _msg_spec(last=D):
    return pl.BlockSpec((2, _BLK, last), lambda i: (0, i, 0))


def _tc1_body(x_ref, wp_ref, bp_ref, o_ref):
    o_ref[...] = jnp.maximum(
        jnp.dot(x_ref[...], wp_ref[...], preferred_element_type=jnp.float32)
        + bp_ref[...], 0.0)


_tc1 = pl.pallas_call(
    _tc1_body,
    grid=(_GRID,),
    in_specs=[_row_spec(), _w_spec(), _b_spec()],
    out_specs=_row_spec(),
    out_shape=jax.ShapeDtypeStruct((NPAD, D), jnp.float32),
)


def _mean(m_ref, c_ref):
    cnt = c_ref[0] + c_ref[1]
    inv = 1.0 / jnp.maximum(cnt[:, 0:1], 1.0)
    return (m_ref[0] + m_ref[1]) * inv


def _tc2_body(m_ref, c_ref, x_ref, wl_ref, bl_ref, wr_ref, wp_ref, bp_ref,
              hmid_ref, h2p_ref):
    mean = _mean(m_ref, c_ref)
    hmid = jnp.maximum(
        jnp.dot(mean, wl_ref[...], preferred_element_type=jnp.float32)
        + bl_ref[...]
        + jnp.dot(x_ref[...], wr_ref[...], preferred_element_type=jnp.float32),
        0.0)
    hmid_ref[...] = hmid
    h2p_ref[...] = jnp.maximum(
        jnp.dot(hmid, wp_ref[...], preferred_element_type=jnp.float32)
        + bp_ref[...], 0.0)


_tc2 = pl.pallas_call(
    _tc2_body,
    grid=(_GRID,),
    in_specs=[_msg_spec(), _msg_spec(), _row_spec(),
              _w_spec(), _b_spec(), _w_spec(), _w_spec(), _b_spec()],
    out_specs=[_row_spec(), _row_spec()],
    out_shape=[jax.ShapeDtypeStruct((NPAD, D), jnp.float32),
               jax.ShapeDtypeStruct((NPAD, D), jnp.float32)],
)


def _tc3_body(m_ref, c_ref, h_ref, wl_ref, bl_ref, wr_ref, wf_ref, bf_ref,
              o_ref):
    mean = _mean(m_ref, c_ref)
    h4 = jnp.maximum(
        jnp.dot(mean, wl_ref[...], preferred_element_type=jnp.float32)
        + bl_ref[...]
        + jnp.dot(h_ref[...], wr_ref[...], preferred_element_type=jnp.float32),
        0.0)
    o_ref[...] = (
        jnp.dot(h4, wf_ref[...], preferred_element_type=jnp.float32)
        + bf_ref[...])


_tc3 = pl.pallas_call(
    _tc3_body,
    grid=(_GRID,),
    in_specs=[_msg_spec(), _msg_spec(), _row_spec(),
              _w_spec(), _b_spec(), _w_spec(), _w_spec(), _b_spec()],
    out_specs=_row_spec(),
    out_shape=jax.ShapeDtypeStruct((NPAD, D), jnp.float32),
)


def kernel(x, edge_index, map_id, node_id,
           Wp1, bp1, Wl1, bl1, Wr1,
           Wp2, bp2, Wl2, bl2, Wr2,
           Wf, bf):
    xp = jnp.pad(x, ((0, NPAD - N), (0, 0)))
    src = edge_index[0]
    dst = edge_index[1]
    npad_e = EPAD - E
    srcp = jnp.concatenate([src, jnp.zeros((npad_e,), jnp.int32)])
    dstp = jnp.concatenate(
        [dst, N + (jnp.arange(npad_e, dtype=jnp.int32) % (NPAD - N))])
    z128 = jnp.zeros((CH, D), jnp.float32)
    o128 = jnp.ones((CH, D), jnp.float32)

    cnt = _make_seg(False)(dstp, z128, o128)
    h1p = _tc1(xp, Wp1.T, bp1.reshape(1, D))
    msg1 = _make_seg(True)(h1p, srcp, dstp, z128)
    hmid, h2p = _tc2(msg1, cnt, xp, Wl1.T, bl1.reshape(1, D), Wr1.T,
                     Wp2.T, bp2.reshape(1, D))
    msg2 = _make_seg(True)(h2p, srcp, dstp, z128)
    hfin = _tc3(msg2, cnt, hmid, Wl2.T, bl2.reshape(1, D), Wr2.T,
                Wf.T, bf.reshape(1, D))
    flat_idx = (
        jnp.arange(BS, dtype=jnp.int32)[:, None] * (N // BS) + map_id
    ).reshape(-1)
    local_feat = _local_gather(hfin, flat_idx)
    return (local_feat, hfin[:N])


# speedup vs baseline: 3.4925x; 1.2142x over previous
import functools

import jax
import jax.numpy as jnp
from jax import lax
from jax.experimental import pallas as pl
from jax.experimental.pallas import tpu as pltpu
from jax.experimental.pallas import tpu_sc as plsc

N = 10000
NPAD = 10240
D = 128
E = 320000
CH = 128
NTILES = 32
CPT = 80
NCHUNK = NTILES * CPT
EPAD = NCHUNK * CH
RPT = NPAD // 16
BS = 16
MAP_LEN = 128

_mesh = plsc.VectorSubcoreMesh(
    core_axis_name="c", subcore_axis_name="s", num_cores=2, num_subcores=16)


def _zero_acc(z128_hbm, rows, acc_sh, s):
    pltpu.sync_copy(z128_hbm, rows)
    for k in range(RPT // CH):
        pltpu.sync_copy(rows, acc_sh.at[pl.ds(s * RPT + k * CH, CH)])


def _msg_body(h_hbm, src_hbm, dst_hbm, z128_hbm, acc_out,
              sv0, sv1, dv0, dv1, rows0, rows1, acc_sh, isem, gsem):
    c = lax.axis_index("c")
    s = lax.axis_index("s")
    wid = s * 2 + c
    base0 = wid * CPT * CH
    sv = (sv0, sv1)
    dv = (dv0, dv1)
    rows = (rows0, rows1)

    _zero_acc(z128_hbm, rows0, acc_sh, s)
    plsc.subcore_barrier()

    def issue_idx(cix, p):
        a = pltpu.async_copy(src_hbm.at[pl.ds(base0 + cix * CH, CH)], sv[p], isem)
        b = pltpu.async_copy(dst_hbm.at[pl.ds(base0 + cix * CH, CH)], dv[p], isem)
        return a, b

    def wait_idx(p):
        pltpu.make_async_copy(src_hbm.at[pl.ds(0, CH)], sv[p], isem).wait()
        pltpu.make_async_copy(dst_hbm.at[pl.ds(0, CH)], dv[p], isem).wait()

    def issue_gather(p):
        return pltpu.async_copy(h_hbm.at[sv[p]], rows[p], gsem)

    def wait_gather(p):
        pltpu.make_async_copy(h_hbm.at[pl.ds(0, CH)], rows[p], gsem).wait()

    def scatter(p):
        pltpu.sync_copy(rows[p], acc_sh.at[dv[p]], add=True)

    a0, b0 = issue_idx(0, 0)
    a0.wait()
    b0.wait()
    issue_gather(0)
    issue_idx(1, 1)

    def body(t, carry):
        wait_idx(1)
        issue_gather(1)
        wait_gather(0)
        scatter(0)
        issue_idx(2 * t + 2, 0)
        wait_idx(0)
        issue_gather(0)
        wait_gather(1)
        scatter(1)
        issue_idx(2 * t + 3, 1)
        return carry

    lax.fori_loop(0, CPT // 2 - 1, body, 0)
    wait_idx(1)
    issue_gather(1)
    wait_gather(0)
    scatter(0)
    wait_gather(1)
    scatter(1)
    plsc.subcore_barrier()

    for k in range(RPT // CH):
        r0 = s * RPT + k * CH
        pltpu.sync_copy(acc_sh.at[pl.ds(r0, CH)], rows0)
        pltpu.sync_copy(rows0, acc_out.at[c, pl.ds(r0, CH)])


_msg_seg = pl.kernel(
    _msg_body,
    out_type=jax.ShapeDtypeStruct((2, NPAD, D), jnp.float32),
    mesh=_mesh,
    scratch_types=[
        pltpu.VMEM((CH,), jnp.int32),
        pltpu.VMEM((CH,), jnp.int32),
        pltpu.VMEM((CH,), jnp.int32),
        pltpu.VMEM((CH,), jnp.int32),
        pltpu.VMEM((CH, D), jnp.float32),
        pltpu.VMEM((CH, D), jnp.float32),
        pltpu.VMEM_SHARED((NPAD, D), jnp.float32),
        pltpu.SemaphoreType.DMA,
        pltpu.SemaphoreType.DMA,
    ],
)


def _cnt_body(dst_hbm, z128_hbm, o128_hbm, acc_out,
              dv0, dv1, rows, acc_sh, isem):
    c = lax.axis_index("c")
    s = lax.axis_index("s")
    wid = s * 2 + c
    base0 = wid * CPT * CH
    dv = (dv0, dv1)

    _zero_acc(z128_hbm, rows, acc_sh, s)
    pltpu.sync_copy(o128_hbm, rows)
    plsc.subcore_barrier()

    def issue_idx(cix, p):
        pltpu.async_copy(dst_hbm.at[pl.ds(base0 + cix * CH, CH)], dv[p], isem)

    def wait_idx(p):
        pltpu.make_async_copy(dst_hbm.at[pl.ds(0, CH)], dv[p], isem).wait()

    issue_idx(0, 0)
    issue_idx(1, 1)

    def body(t, carry):
        wait_idx(0)
        pltpu.sync_copy(rows, acc_sh.at[dv0], add=True)
        issue_idx(2 * t + 2, 0)
        wait_idx(1)
        pltpu.sync_copy(rows, acc_sh.at[dv1], add=True)
        issue_idx(2 * t + 3, 1)
        return carry

    lax.fori_loop(0, CPT // 2 - 1, body, 0)
    wait_idx(0)
    pltpu.sync_copy(rows, acc_sh.at[dv0], add=True)
    wait_idx(1)
    pltpu.sync_copy(rows, acc_sh.at[dv1], add=True)
    plsc.subcore_barrier()

    for k in range(RPT // CH):
        r0 = s * RPT + k * CH
        pltpu.sync_copy(acc_sh.at[pl.ds(r0, CH)], rows)
        pltpu.sync_copy(rows, acc_out.at[c, pl.ds(r0, CH)])


_cnt_seg = pl.kernel(
    _cnt_body,
    out_type=jax.ShapeDtypeStruct((2, NPAD, D), jnp.float32),
    mesh=_mesh,
    scratch_types=[
        pltpu.VMEM((CH,), jnp.int32),
        pltpu.VMEM((CH,), jnp.int32),
        pltpu.VMEM((CH, D), jnp.float32),
        pltpu.VMEM_SHARED((NPAD, D), jnp.float32),
        pltpu.SemaphoreType.DMA,
    ],
)


def _gather_body(h_hbm, idx_hbm, out_hbm, idxv, rows, sem):
    c = lax.axis_index("c")
    s = lax.axis_index("s")
    base = s * MAP_LEN + c * 64
    pltpu.sync_copy(idx_hbm.at[pl.ds(base, 64)], idxv)
    pltpu.async_copy(h_hbm.at[idxv], rows, sem).wait()
    pltpu.sync_copy(rows, out_hbm.at[pl.ds(base, 64)])


_local_gather = pl.kernel(
    _gather_body,
    out_type=jax.ShapeDtypeStruct((BS * MAP_LEN, D), jnp.float32),
    mesh=_mesh,
    scratch_types=[
        pltpu.VMEM((64,), jnp.int32),
        pltpu.VMEM((64, D), jnp.float32),
        pltpu.SemaphoreType.DMA,
    ],
)


_BLK = 512
_GRID = NPAD // _BLK


def _row_spec(last=D):
    return pl.BlockSpec((_BLK, last), lambda i: (i, 0))


def _w_spec():
    return pl.BlockSpec((D, D), lambda i: (0, 0))


def _b_spec():
    return pl.BlockSpec((1, D), lambda i: (0, 0))


def _msg_spec(last=D):
    return pl.BlockSpec((2, _BLK, last), lambda i: (0, i, 0))


def _tc1_body(x_ref, wp_ref, bp_ref, o_ref):
    o_ref[...] = jnp.maximum(
        jnp.dot(x_ref[...], wp_ref[...], preferred_element_type=jnp.float32)
        + bp_ref[...], 0.0)


_tc1 = pl.pallas_call(
    _tc1_body,
    grid=(_GRID,),
    in_specs=[_row_spec(), _w_spec(), _b_spec()],
    out_specs=_row_spec(),
    out_shape=jax.ShapeDtypeStruct((NPAD, D), jnp.float32),
)


def _mean(m_ref, c_ref):
    cnt = c_ref[0] + c_ref[1]
    inv = 1.0 / jnp.maximum(cnt[:, 0:1], 1.0)
    return (m_ref[0] + m_ref[1]) * inv


def _tc2_body(m_ref, c_ref, x_ref, wl_ref, bl_ref, wr_ref, wp_ref, bp_ref,
              hmid_ref, h2p_ref):
    mean = _mean(m_ref, c_ref)
    hmid = jnp.maximum(
        jnp.dot(mean, wl_ref[...], preferred_element_type=jnp.float32)
        + bl_ref[...]
        + jnp.dot(x_ref[...], wr_ref[...], preferred_element_type=jnp.float32),
        0.0)
    hmid_ref[...] = hmid
    h2p_ref[...] = jnp.maximum(
        jnp.dot(hmid, wp_ref[...], preferred_element_type=jnp.float32)
        + bp_ref[...], 0.0)


_tc2 = pl.pallas_call(
    _tc2_body,
    grid=(_GRID,),
    in_specs=[_msg_spec(), _msg_spec(), _row_spec(),
              _w_spec(), _b_spec(), _w_spec(), _w_spec(), _b_spec()],
    out_specs=[_row_spec(), _row_spec()],
    out_shape=[jax.ShapeDtypeStruct((NPAD, D), jnp.float32),
               jax.ShapeDtypeStruct((NPAD, D), jnp.float32)],
)


def _tc3_body(m_ref, c_ref, h_ref, wl_ref, bl_ref, wr_ref, wf_ref, bf_ref,
              o_ref):
    mean = _mean(m_ref, c_ref)
    h4 = jnp.maximum(
        jnp.dot(mean, wl_ref[...], preferred_element_type=jnp.float32)
        + bl_ref[...]
        + jnp.dot(h_ref[...], wr_ref[...], preferred_element_type=jnp.float32),
        0.0)
    o_ref[...] = (
        jnp.dot(h4, wf_ref[...], preferred_element_type=jnp.float32)
        + bf_ref[...])


_tc3 = pl.pallas_call(
    _tc3_body,
    grid=(_GRID,),
    in_specs=[_msg_spec(), _msg_spec(), _row_spec(),
              _w_spec(), _b_spec(), _w_spec(), _w_spec(), _b_spec()],
    out_specs=_row_spec(),
    out_shape=jax.ShapeDtypeStruct((NPAD, D), jnp.float32),
)


def kernel(x, edge_index, map_id, node_id,
           Wp1, bp1, Wl1, bl1, Wr1,
           Wp2, bp2, Wl2, bl2, Wr2,
           Wf, bf):
    xp = jnp.pad(x, ((0, NPAD - N), (0, 0)))
    src = edge_index[0]
    dst = edge_index[1]
    npad_e = EPAD - E
    srcp = jnp.concatenate([src, jnp.zeros((npad_e,), jnp.int32)])
    dstp = jnp.concatenate(
        [dst, N + (jnp.arange(npad_e, dtype=jnp.int32) % (NPAD - N))])
    z128 = jnp.zeros((CH, D), jnp.float32)
    o128 = jnp.ones((CH, D), jnp.float32)

    cnt = _cnt_seg(dstp, z128, o128)
    h1p = _tc1(xp, Wp1.T, bp1.reshape(1, D))
    msg1 = _msg_seg(h1p, srcp, dstp, z128)
    hmid, h2p = _tc2(msg1, cnt, xp, Wl1.T, bl1.reshape(1, D), Wr1.T,
                     Wp2.T, bp2.reshape(1, D))
    msg2 = _msg_seg(h2p, srcp, dstp, z128)
    hfin = _tc3(msg2, cnt, hmid, Wl2.T, bl2.reshape(1, D), Wr2.T,
                Wf.T, bf.reshape(1, D))
    flat_idx = (
        jnp.arange(BS, dtype=jnp.int32)[:, None] * (N // BS) + map_id
    ).reshape(-1)
    local_feat = _local_gather(hfin, flat_idx)
    return (local_feat, hfin[:N])


# speedup vs baseline: 3.6515x; 1.0455x over previous
import functools

import jax
import jax.numpy as jnp
from jax import lax
from jax.experimental import pallas as pl
from jax.experimental.pallas import tpu as pltpu
from jax.experimental.pallas import tpu_sc as plsc

N = 10000
NPAD = 10240
D = 128
E = 320000
CH = 128
NTILES = 32
CPT = 80
CPT0 = 128
CPT1 = 32
NCHUNK = NTILES * CPT
EPAD = NCHUNK * CH
RPT = NPAD // 16
BS = 16
MAP_LEN = 128

_mesh = plsc.VectorSubcoreMesh(
    core_axis_name="c", subcore_axis_name="s", num_cores=2, num_subcores=16)


def _zero_acc(z128_hbm, rows, acc_sh, s):
    pltpu.sync_copy(z128_hbm, rows)
    for k in range(RPT // CH):
        pltpu.sync_copy(rows, acc_sh.at[pl.ds(s * RPT + k * CH, CH)])


def _msg_body(h_hbm, src_hbm, dst_hbm, z128_hbm, acc_out,
              sv0, sv1, dv0, dv1, rows0, rows1, acc_sh, isem, gsem):
    c = lax.axis_index("c")
    s = lax.axis_index("s")
    nchunk = jnp.where(c == 0, CPT0, CPT1)
    base0 = jnp.where(c == 0, s * CPT0, 16 * CPT0 + s * CPT1) * CH
    sv = (sv0, sv1)
    dv = (dv0, dv1)
    rows = (rows0, rows1)

    _zero_acc(z128_hbm, rows0, acc_sh, s)
    plsc.subcore_barrier()

    def issue_idx(cix, p):
        a = pltpu.async_copy(src_hbm.at[pl.ds(base0 + cix * CH, CH)], sv[p], isem)
        b = pltpu.async_copy(dst_hbm.at[pl.ds(base0 + cix * CH, CH)], dv[p], isem)
        return a, b

    def wait_idx(p):
        pltpu.make_async_copy(src_hbm.at[pl.ds(0, CH)], sv[p], isem).wait()
        pltpu.make_async_copy(dst_hbm.at[pl.ds(0, CH)], dv[p], isem).wait()

    def issue_gather(p):
        return pltpu.async_copy(h_hbm.at[sv[p]], rows[p], gsem)

    def wait_gather(p):
        pltpu.make_async_copy(h_hbm.at[pl.ds(0, CH)], rows[p], gsem).wait()

    def scatter(p):
        pltpu.sync_copy(rows[p], acc_sh.at[dv[p]], add=True)

    @pl.when(nchunk > 0)
    def _run():
        a0, b0 = issue_idx(0, 0)
        a0.wait()
        b0.wait()
        issue_gather(0)
        issue_idx(1, 1)

        def body(t, carry):
            wait_idx(1)
            issue_gather(1)
            wait_gather(0)
            scatter(0)
            issue_idx(2 * t + 2, 0)
            wait_idx(0)
            issue_gather(0)
            wait_gather(1)
            scatter(1)
            issue_idx(2 * t + 3, 1)
            return carry

        lax.fori_loop(0, nchunk // 2 - 1, body, 0)
        wait_idx(1)
        issue_gather(1)
        wait_gather(0)
        scatter(0)
        wait_gather(1)
        scatter(1)

    plsc.subcore_barrier()

    for k in range(RPT // CH):
        r0 = s * RPT + k * CH
        pltpu.sync_copy(acc_sh.at[pl.ds(r0, CH)], rows0)
        pltpu.sync_copy(rows0, acc_out.at[c, pl.ds(r0, CH)])


_msg_seg = pl.kernel(
    _msg_body,
    out_type=jax.ShapeDtypeStruct((2, NPAD, D), jnp.float32),
    mesh=_mesh,
    scratch_types=[
        pltpu.VMEM((CH,), jnp.int32),
        pltpu.VMEM((CH,), jnp.int32),
        pltpu.VMEM((CH,), jnp.int32),
        pltpu.VMEM((CH,), jnp.int32),
        pltpu.VMEM((CH, D), jnp.float32),
        pltpu.VMEM((CH, D), jnp.float32),
        pltpu.VMEM_SHARED((NPAD, D), jnp.float32),
        pltpu.SemaphoreType.DMA,
        pltpu.SemaphoreType.DMA,
    ],
)


def _cnt_body(dst_hbm, z128_hbm, o128_hbm, acc_out,
              dv0, dv1, rows, acc_sh, isem):
    c = lax.axis_index("c")
    s = lax.axis_index("s")
    wid = s * 2 + c
    base0 = wid * CPT * CH
    dv = (dv0, dv1)

    _zero_acc(z128_hbm, rows, acc_sh, s)
    pltpu.sync_copy(o128_hbm, rows)
    plsc.subcore_barrier()

    def issue_idx(cix, p):
        pltpu.async_copy(dst_hbm.at[pl.ds(base0 + cix * CH, CH)], dv[p], isem)

    def wait_idx(p):
        pltpu.make_async_copy(dst_hbm.at[pl.ds(0, CH)], dv[p], isem).wait()

    issue_idx(0, 0)
    issue_idx(1, 1)

    def body(t, carry):
        wait_idx(0)
        pltpu.sync_copy(rows, acc_sh.at[dv0], add=True)
        issue_idx(2 * t + 2, 0)
        wait_idx(1)
        pltpu.sync_copy(rows, acc_sh.at[dv1], add=True)
        issue_idx(2 * t + 3, 1)
        return carry

    lax.fori_loop(0, CPT // 2 - 1, body, 0)
    wait_idx(0)
    pltpu.sync_copy(rows, acc_sh.at[dv0], add=True)
    wait_idx(1)
    pltpu.sync_copy(rows, acc_sh.at[dv1], add=True)
    plsc.subcore_barrier()

    for k in range(RPT // CH):
        r0 = s * RPT + k * CH
        pltpu.sync_copy(acc_sh.at[pl.ds(r0, CH)], rows)
        pltpu.sync_copy(rows, acc_out.at[c, pl.ds(r0, CH)])


_cnt_seg = pl.kernel(
    _cnt_body,
    out_type=jax.ShapeDtypeStruct((2, NPAD, D), jnp.float32),
    mesh=_mesh,
    scratch_types=[
        pltpu.VMEM((CH,), jnp.int32),
        pltpu.VMEM((CH,), jnp.int32),
        pltpu.VMEM((CH, D), jnp.float32),
        pltpu.VMEM_SHARED((NPAD, D), jnp.float32),
        pltpu.SemaphoreType.DMA,
    ],
)


def _gather_body(h_hbm, idx_hbm, out_hbm, idxv, rows, sem):
    c = lax.axis_index("c")
    s = lax.axis_index("s")
    base = s * MAP_LEN + c * 64
    pltpu.sync_copy(idx_hbm.at[pl.ds(base, 64)], idxv)
    pltpu.async_copy(h_hbm.at[idxv], rows, sem).wait()
    pltpu.sync_copy(rows, out_hbm.at[pl.ds(base, 64)])


_local_gather = pl.kernel(
    _gather_body,
    out_type=jax.ShapeDtypeStruct((BS * MAP_LEN, D), jnp.float32),
    mesh=_mesh,
    scratch_types=[
        pltpu.VMEM((64,), jnp.int32),
        pltpu.VMEM((64, D), jnp.float32),
        pltpu.SemaphoreType.DMA,
    ],
)


_BLK = 512
_GRID = NPAD // _BLK


def _row_spec(last=D):
    return pl.BlockSpec((_BLK, last), lambda i: (i, 0))


def _w_spec():
    return pl.BlockSpec((D, D), lambda i: (0, 0))


def _b_spec():
    return pl.BlockSpec((1, D), lambda i: (0, 0))


def _msg_spec(last=D):
    return pl.BlockSpec((2, _BLK, last), lambda i: (0, i, 0))


def _tc1_body(x_ref, wp_ref, bp_ref, o_ref):
    o_ref[...] = jnp.maximum(
        jnp.dot(x_ref[...], wp_ref[...], preferred_element_type=jnp.float32)
        + bp_ref[...], 0.0)


_tc1 = pl.pallas_call(
    _tc1_body,
    grid=(_GRID,),
    in_specs=[_row_spec(), _w_spec(), _b_spec()],
    out_specs=_row_spec(),
    out_shape=jax.ShapeDtypeStruct((NPAD, D), jnp.float32),
)


def _mean(m_ref, c_ref):
    cnt = c_ref[0] + c_ref[1]
    inv = 1.0 / jnp.maximum(cnt[:, 0:1], 1.0)
    return (m_ref[0] + m_ref[1]) * inv


def _tc2_body(m_ref, c_ref, x_ref, wl_ref, bl_ref, wr_ref, wp_ref, bp_ref,
              hmid_ref, h2p_ref):
    mean = _mean(m_ref, c_ref)
    hmid = jnp.maximum(
        jnp.dot(mean, wl_ref[...], preferred_element_type=jnp.float32)
        + bl_ref[...]
        + jnp.dot(x_ref[...], wr_ref[...], preferred_element_type=jnp.float32),
        0.0)
    hmid_ref[...] = hmid
    h2p_ref[...] = jnp.maximum(
        jnp.dot(hmid, wp_ref[...], preferred_element_type=jnp.float32)
        + bp_ref[...], 0.0)


_tc2 = pl.pallas_call(
    _tc2_body,
    grid=(_GRID,),
    in_specs=[_msg_spec(), _msg_spec(), _row_spec(),
              _w_spec(), _b_spec(), _w_spec(), _w_spec(), _b_spec()],
    out_specs=[_row_spec(), _row_spec()],
    out_shape=[jax.ShapeDtypeStruct((NPAD, D), jnp.float32),
               jax.ShapeDtypeStruct((NPAD, D), jnp.float32)],
)


def _tc3_body(m_ref, c_ref, h_ref, wl_ref, bl_ref, wr_ref, wf_ref, bf_ref,
              o_ref):
    mean = _mean(m_ref, c_ref)
    h4 = jnp.maximum(
        jnp.dot(mean, wl_ref[...], preferred_element_type=jnp.float32)
        + bl_ref[...]
        + jnp.dot(h_ref[...], wr_ref[...], preferred_element_type=jnp.float32),
        0.0)
    o_ref[...] = (
        jnp.dot(h4, wf_ref[...], preferred_element_type=jnp.float32)
        + bf_ref[...])


_tc3 = pl.pallas_call(
    _tc3_body,
    grid=(_GRID,),
    in_specs=[_msg_spec(), _msg_spec(), _row_spec(),
              _w_spec(), _b_spec(), _w_spec(), _w_spec(), _b_spec()],
    out_specs=_row_spec(),
    out_shape=jax.ShapeDtypeStruct((NPAD, D), jnp.float32),
)


def kernel(x, edge_index, map_id, node_id,
           Wp1, bp1, Wl1, bl1, Wr1,
           Wp2, bp2, Wl2, bl2, Wr2,
           Wf, bf):
    xp = jnp.pad(x, ((0, NPAD - N), (0, 0)))
    src = edge_index[0]
    dst = edge_index[1]
    npad_e = EPAD - E
    srcp = jnp.concatenate([src, jnp.zeros((npad_e,), jnp.int32)])
    dstp = jnp.concatenate(
        [dst, N + (jnp.arange(npad_e, dtype=jnp.int32) % (NPAD - N))])
    z128 = jnp.zeros((CH, D), jnp.float32)
    o128 = jnp.ones((CH, D), jnp.float32)

    cnt = _cnt_seg(dstp, z128, o128)
    h1p = _tc1(xp, Wp1.T, bp1.reshape(1, D))
    msg1 = _msg_seg(h1p, srcp, dstp, z128)
    hmid, h2p = _tc2(msg1, cnt, xp, Wl1.T, bl1.reshape(1, D), Wr1.T,
                     Wp2.T, bp2.reshape(1, D))
    msg2 = _msg_seg(h2p, srcp, dstp, z128)
    hfin = _tc3(msg2, cnt, hmid, Wl2.T, bl2.reshape(1, D), Wr2.T,
                Wf.T, bf.reshape(1, D))
    flat_idx = (
        jnp.arange(BS, dtype=jnp.int32)[:, None] * (N // BS) + map_id
    ).reshape(-1)
    local_feat = _local_gather(hfin, flat_idx)
    return (local_feat, hfin[:N])
